# baseline (speedup 1.0000x reference)
import jax
import jax.numpy as jnp
from jax import lax
from jax.experimental import pallas as pl
from jax.experimental.pallas import tpu as pltpu
from jax.experimental.pallas import tpu_sc as plsc

N = 10000
E = 160000
D_IN = 256
D_OUT = 256

NPAD = 10240
NTILES = 32
EPAD = 163840
SENTINEL = NPAD - 1

EPT = EPAD // NTILES
_MESH = plsc.VectorSubcoreMesh(core_axis_name="c", subcore_axis_name="s")
_SC_PARAMS = pltpu.CompilerParams(needs_layout_passes=False)


def _deg_hist_body(dst_hbm, out_hbm, dvm, hist):
    c = lax.axis_index("c")
    s = lax.axis_index("s")
    wid = c * 16 + s

    zero = jnp.zeros((16,), jnp.float32)

    def zbody(j, _):
        hist[pl.ds(j * 16, 16)] = zero
        return 0

    lax.fori_loop(0, NPAD // 16, zbody, 0)

    pltpu.sync_copy(dst_hbm.at[pl.ds(wid * EPT, EPT)], dvm)

    ones = jnp.ones((16,), jnp.float32)

    def body(j, _):
        d = dvm[pl.ds(j * 16, 16)]
        plsc.addupdate_scatter(hist, [d], ones)
        return 0

    lax.fori_loop(0, EPT // 16, body, 0)

    pltpu.sync_copy(hist, out_hbm.at[pl.ds(wid * NPAD, NPAD)])


def _deg_hist(dst_pad):
    return pl.kernel(
        _deg_hist_body,
        out_type=jax.ShapeDtypeStruct((NTILES * NPAD,), jnp.float32),
        mesh=_MESH,
        compiler_params=_SC_PARAMS,
        scratch_types=[
            pltpu.VMEM((EPT,), jnp.int32),
            pltpu.VMEM((NPAD,), jnp.float32),
        ],
    )(dst_pad)


_PER_TILE = 512
_ACTIVE_TILES = NPAD // _PER_TILE


def _dis_body(part_hbm, dis_hbm, pvm, dvm):
    c = lax.axis_index("c")
    s = lax.axis_index("s")
    wid = c * 16 + s
    base = wid * _PER_TILE

    @pl.when(wid < _ACTIVE_TILES)
    def _():
        for r in range(NTILES):
            pltpu.sync_copy(part_hbm.at[pl.ds(r * NPAD + base, _PER_TILE)],
                            pvm.at[r])

        def body(j, _):
            acc = jnp.zeros((16,), jnp.float32)
            for r in range(NTILES):
                acc = acc + pvm[r, pl.ds(j * 16, 16)]
            d = acc + 1.0
            i = plsc.bitcast(d, jnp.int32)
            i = jnp.int32(0x5F3759DF) - (i >> 1)
            y = plsc.bitcast(i, jnp.float32)
            half = d * (-0.5)
            for _it in range(3):
                y = y * (1.5 + half * y * y)
            dvm[pl.ds(j * 16, 16)] = y
            return 0

        lax.fori_loop(0, _PER_TILE // 16, body, 0)
        pltpu.sync_copy(dvm, dis_hbm.at[pl.ds(base, _PER_TILE)])


def _dis_from_parts(parts):
    return pl.kernel(
        _dis_body,
        out_type=jax.ShapeDtypeStruct((NPAD,), jnp.float32),
        mesh=_MESH,
        compiler_params=_SC_PARAMS,
        scratch_types=[
            pltpu.VMEM((NTILES, _PER_TILE), jnp.float32),
            pltpu.VMEM((_PER_TILE,), jnp.float32),
        ],
    )(parts)


NH = N // 2
_ROWS_A = 312
_ROWS_B = NH - 15 * _ROWS_A
ACC_ROWS = _ROWS_B + 8
TRASH = _ROWS_B
ECH = 2048
NSEG = EPAD // ECH
SCAP = ECH + 112
GK = 48


def _scatter_body(src_hbm, dst_hbm, hs_hbm, tmp_hbm,
                  svm0, svm1, dvm0, dvm1, sc0, sc1, rc0, rc1,
                  gbuf, acc, msegs, sem0, sem1, semsa, semsb):
    c = lax.axis_index("c")
    s = lax.axis_index("s")
    base_row = c * NH + s * _ROWS_A
    rcnt = jnp.where(s == 15, _ROWS_B, _ROWS_A)

    @pl.when(s < 15)
    def _():
        pltpu.sync_copy(hs_hbm.at[pl.ds(base_row, _ROWS_A)],
                        acc.at[pl.ds(0, _ROWS_A)])

    @pl.when(s == 15)
    def _():
        pltpu.sync_copy(hs_hbm.at[pl.ds(base_row, _ROWS_B)],
                        acc.at[pl.ds(0, _ROWS_B)])

    pad16 = jnp.zeros((16,), jnp.int32) + base_row
    trash16 = jnp.full((16,), TRASH, jnp.int32)
    stage_sems = (semsa, semsb)
    svm = (svm0, svm1)
    dvm = (dvm0, dvm1)
    src_c = (sc0, sc1)
    rel_c = (rc0, rc1)

    def fire_staging(qs, seg):
        pltpu.async_copy(src_hbm.at[pl.ds(seg * ECH, ECH)], svm[qs],
                         stage_sems[qs])
        pltpu.async_copy(dst_hbm.at[pl.ds(seg * ECH, ECH)], dvm[qs],
                         stage_sems[qs])

    def wait_staging(qs, seg):
        pltpu.make_async_copy(src_hbm.at[pl.ds(seg * ECH, ECH)], svm[qs],
                              stage_sems[qs]).wait()
        pltpu.make_async_copy(dst_hbm.at[pl.ds(seg * ECH, ECH)], dvm[qs],
                              stage_sems[qs]).wait()

    def filter_seg(qs):
        def fbody(j, ptr):
            sv = svm[qs][pl.ds(j * 16, 16)]
            dv = dvm[qs][pl.ds(j * 16, 16)]
            rel = dv - base_row
            m = (rel >= 0) & (rel < rcnt)
            plsc.store_compressed(src_c[qs].at[pl.ds(ptr, 16)], sv, mask=m)
            plsc.store_compressed(rel_c[qs].at[pl.ds(ptr, 16)], rel, mask=m)
            return ptr + jnp.sum(jnp.where(m, 1, 0).astype(jnp.int32))

        mseg = lax.fori_loop(0, ECH // 16, fbody, jnp.int32(0))
        ngrp = (mseg + (GK - 1)) // GK

        def pbody(j, _):
            off = mseg + j * 16
            src_c[qs][pl.ds(off, 16)] = pad16
            rel_c[qs][pl.ds(off, 16)] = trash16
            return 0

        lax.fori_loop(0, (ngrp * GK - mseg + 15) // 16, pbody, 0)
        msegs[qs] = mseg

    def fire_gather(qs, k, p):
        sem = sem1 if p else sem0
        pltpu.async_copy(hs_hbm.at[src_c[qs].at[pl.ds(k * GK, GK)]],
                         gbuf.at[p], sem)

    def wait_gather(qs, p):
        sem = sem1 if p else sem0
        pltpu.make_async_copy(hs_hbm.at[src_c[qs].at[pl.ds(0, GK)]],
                              gbuf.at[p], sem).wait()

    def adds_seg(qs):
        mseg = msegs[qs]
        ngrp = (mseg + (GK - 1)) // GK

        def gbody(k, _):
            p = lax.rem(k, 2)

            @pl.when((k + 1 < ngrp) & (p == 0))
            def _():
                fire_gather(qs, k + 1, 1)

            @pl.when((k + 1 < ngrp) & (p == 1))
            def _():
                fire_gather(qs, k + 1, 0)

            @pl.when(p == 0)
            def _():
                wait_gather(qs, 0)

            @pl.when(p == 1)
            def _():
                wait_gather(qs, 1)

            def group(g, _):
                relv = rel_c[qs][pl.ds(k * GK + g * 16, 16)]
                for i in range(16):
                    rel = relv[i]
                    for j in range(D_OUT // 16):
                        plsc.addupdate(
                            acc.at[rel, pl.ds(j * 16, 16)],
                            gbuf[p, g * 16 + i, pl.ds(j * 16, 16)])
                return 0

            lax.fori_loop(0, GK // 16, group, 0)
            return 0

        lax.fori_loop(0, ngrp, gbody, 0)

    pltpu.sync_copy(src_hbm.at[pl.ds(0, ECH)], svm[0])
    pltpu.sync_copy(dst_hbm.at[pl.ds(0, ECH)], dvm[0])
    filter_seg(0)
    fire_staging(1, 1)

    @pl.when(msegs[0] > 0)
    def _():
        fire_gather(0, 0, 0)

    def segment(seg, _):
        q = lax.rem(seg, 2)

        def step(qs):
            @pl.when(seg + 1 < NSEG)
            def _():
                wait_staging(1 - qs, seg + 1)
                filter_seg(1 - qs)

                @pl.when(seg + 2 < NSEG)
                def _():
                    fire_staging(qs, seg + 2)

            adds_seg(qs)

            @pl.when((seg + 1 < NSEG) & (msegs[1 - qs] > 0))
            def _():
                fire_gather(1 - qs, 0, 0)

        @pl.when(q == 0)
        def _():
            step(0)

        @pl.when(q == 1)
        def _():
            step(1)

        return 0

    lax.fori_loop(0, NSEG, segment, 0)

    @pl.when(s < 15)
    def _():
        pltpu.sync_copy(acc.at[pl.ds(0, _ROWS_A)],
                        tmp_hbm.at[pl.ds(base_row, _ROWS_A)])

    @pl.when(s == 15)
    def _():
        pltpu.sync_copy(acc.at[pl.ds(0, _ROWS_B)],
                        tmp_hbm.at[pl.ds(base_row, _ROWS_B)])


def _scatter_add(src_pad, dst_pad, hs):
    return pl.kernel(
        _scatter_body,
        out_type=jax.ShapeDtypeStruct((N, D_OUT), jnp.float32),
        mesh=_MESH,
        compiler_params=_SC_PARAMS,
        scratch_types=[
            pltpu.VMEM((ECH,), jnp.int32),
            pltpu.VMEM((ECH,), jnp.int32),
            pltpu.VMEM((ECH,), jnp.int32),
            pltpu.VMEM((ECH,), jnp.int32),
            pltpu.VMEM((SCAP,), jnp.int32),
            pltpu.VMEM((SCAP,), jnp.int32),
            pltpu.VMEM((SCAP,), jnp.int32),
            pltpu.VMEM((SCAP,), jnp.int32),
            pltpu.VMEM((2, GK, D_OUT), jnp.float32),
            pltpu.VMEM((ACC_ROWS, D_OUT), jnp.float32),
            pltpu.SMEM((2,), jnp.int32),
            pltpu.SemaphoreType.DMA,
            pltpu.SemaphoreType.DMA,
            pltpu.SemaphoreType.DMA,
            pltpu.SemaphoreType.DMA,
        ],
    )(src_pad, dst_pad, hs)


_BM = 2048


def _mm_body(x_ref, w_ref, dis_ref, o_ref):
    o_ref[...] = jnp.dot(x_ref[...], w_ref[...],
                         preferred_element_type=jnp.float32) * dis_ref[...]


def _matmul_scale(x, W, dis_col):
    M, K = x.shape
    _, Nc = W.shape
    return pl.pallas_call(
        _mm_body,
        grid=(pl.cdiv(M, _BM),),
        in_specs=[pl.BlockSpec((_BM, K), lambda i: (i, 0)),
                  pl.BlockSpec((K, Nc), lambda i: (0, 0)),
                  pl.BlockSpec((_BM, 1), lambda i: (i, 0))],
        out_specs=pl.BlockSpec((_BM, Nc), lambda i: (i, 0)),
        out_shape=jax.ShapeDtypeStruct((M, Nc), jnp.float32),
    )(x, W, dis_col)


def _elu_body(t_ref, dis_ref, b_ref, o_ref):
    v = t_ref[...] * dis_ref[...] + b_ref[...]
    o_ref[...] = jnp.where(v > 0, v, jnp.exp(v) - 1.0)


def _scale_bias_elu(tmp, dis_col, b2d):
    M, Nc = tmp.shape
    return pl.pallas_call(
        _elu_body,
        grid=(pl.cdiv(M, _BM),),
        in_specs=[pl.BlockSpec((_BM, Nc), lambda i: (i, 0)),
                  pl.BlockSpec((_BM, 1), lambda i: (i, 0)),
                  pl.BlockSpec((1, Nc), lambda i: (0, 0))],
        out_specs=pl.BlockSpec((_BM, Nc), lambda i: (i, 0)),
        out_shape=jax.ShapeDtypeStruct((M, Nc), jnp.float32),
    )(tmp, dis_col, b2d)


def kernel(x, edge_index, W, b):
    src = edge_index[0].astype(jnp.int32)
    dst = edge_index[1].astype(jnp.int32)
    pad_n = EPAD - E
    src_pad = jnp.concatenate([src, jnp.zeros((pad_n,), jnp.int32)])
    dst_pad = jnp.concatenate([dst, jnp.full((pad_n,), SENTINEL, jnp.int32)])

    parts = _deg_hist(dst_pad)
    dis = _dis_from_parts(parts)
    dis_col = dis[:N].reshape(N, 1)

    hs = _matmul_scale(x, W, dis_col)
    tmp = _scatter_add(src_pad, dst_pad, hs)
    return _scale_bias_elu(tmp, dis_col, b.reshape(1, D_OUT))


# speedup vs baseline: 1.0517x; 1.0517x over previous
import jax
import jax.numpy as jnp
from jax import lax
from jax.experimental import pallas as pl
from jax.experimental.pallas import tpu as pltpu
from jax.experimental.pallas import tpu_sc as plsc

N = 10000
E = 160000
D_IN = 256
D_OUT = 256

NPAD = 10240
NTILES = 32
EPAD = 163840
SENTINEL = NPAD - 1

EPT = EPAD // NTILES
_MESH = plsc.VectorSubcoreMesh(core_axis_name="c", subcore_axis_name="s")
_SC_PARAMS = pltpu.CompilerParams(needs_layout_passes=False)


def _deg_hist_body(dst_hbm, out_hbm, dvm, hist):
    c = lax.axis_index("c")
    s = lax.axis_index("s")
    wid = c * 16 + s

    zero = jnp.zeros((16,), jnp.float32)

    def zbody(j, _):
        hist[pl.ds(j * 16, 16)] = zero
        return 0

    lax.fori_loop(0, NPAD // 16, zbody, 0)

    pltpu.sync_copy(dst_hbm.at[pl.ds(wid * EPT, EPT)], dvm)

    ones = jnp.ones((16,), jnp.float32)

    def body(j, _):
        d = dvm[pl.ds(j * 16, 16)]
        plsc.addupdate_scatter(hist, [d], ones)
        return 0

    lax.fori_loop(0, EPT // 16, body, 0)

    pltpu.sync_copy(hist, out_hbm.at[pl.ds(wid * NPAD, NPAD)])


def _deg_hist(dst_pad):
    return pl.kernel(
        _deg_hist_body,
        out_type=jax.ShapeDtypeStruct((NTILES * NPAD,), jnp.float32),
        mesh=_MESH,
        compiler_params=_SC_PARAMS,
        scratch_types=[
            pltpu.VMEM((EPT,), jnp.int32),
            pltpu.VMEM((NPAD,), jnp.float32),
        ],
    )(dst_pad)


_PER_TILE = 512
_ACTIVE_TILES = NPAD // _PER_TILE


def _dis_body(part_hbm, dis_hbm, pvm, dvm):
    c = lax.axis_index("c")
    s = lax.axis_index("s")
    wid = c * 16 + s
    base = wid * _PER_TILE

    @pl.when(wid < _ACTIVE_TILES)
    def _():
        for r in range(NTILES):
            pltpu.sync_copy(part_hbm.at[pl.ds(r * NPAD + base, _PER_TILE)],
                            pvm.at[r])

        def body(j, _):
            acc = jnp.zeros((16,), jnp.float32)
            for r in range(NTILES):
                acc = acc + pvm[r, pl.ds(j * 16, 16)]
            d = acc + 1.0
            i = plsc.bitcast(d, jnp.int32)
            i = jnp.int32(0x5F3759DF) - (i >> 1)
            y = plsc.bitcast(i, jnp.float32)
            half = d * (-0.5)
            for _it in range(3):
                y = y * (1.5 + half * y * y)
            dvm[pl.ds(j * 16, 16)] = y
            return 0

        lax.fori_loop(0, _PER_TILE // 16, body, 0)
        pltpu.sync_copy(dvm, dis_hbm.at[pl.ds(base, _PER_TILE)])


def _dis_from_parts(parts):
    return pl.kernel(
        _dis_body,
        out_type=jax.ShapeDtypeStruct((NPAD,), jnp.float32),
        mesh=_MESH,
        compiler_params=_SC_PARAMS,
        scratch_types=[
            pltpu.VMEM((NTILES, _PER_TILE), jnp.float32),
            pltpu.VMEM((_PER_TILE,), jnp.float32),
        ],
    )(parts)


NH = N // 2
_ROWS_A = 312
_ROWS_B = NH - 15 * _ROWS_A
ACC_ROWS = _ROWS_B + 8
TRASH = _ROWS_B
ECH = 5120
NSEG = EPAD // ECH
SCAP = ECH + 112
GK = 48


def _scatter_body(src_hbm, dst_hbm, hs_hbm, tmp_hbm,
                  svm, dvm, src_c, rel_c, gbuf, acc, sem0, sem1):
    c = lax.axis_index("c")
    s = lax.axis_index("s")
    base_row = c * NH + s * _ROWS_A
    rcnt = jnp.where(s == 15, _ROWS_B, _ROWS_A)

    @pl.when(s < 15)
    def _():
        pltpu.sync_copy(hs_hbm.at[pl.ds(base_row, _ROWS_A)],
                        acc.at[pl.ds(0, _ROWS_A)])

    @pl.when(s == 15)
    def _():
        pltpu.sync_copy(hs_hbm.at[pl.ds(base_row, _ROWS_B)],
                        acc.at[pl.ds(0, _ROWS_B)])

    pad16 = jnp.zeros((16,), jnp.int32) + base_row
    trash16 = jnp.full((16,), TRASH, jnp.int32)

    def segment(seg, _):
        pltpu.sync_copy(src_hbm.at[pl.ds(seg * ECH, ECH)], svm)
        pltpu.sync_copy(dst_hbm.at[pl.ds(seg * ECH, ECH)], dvm)

        def fbody(j, ptr):
            sv = svm[pl.ds(j * 16, 16)]
            dv = dvm[pl.ds(j * 16, 16)]
            rel = dv - base_row
            m = (rel >= 0) & (rel < rcnt)
            plsc.store_compressed(src_c.at[pl.ds(ptr, 16)], sv, mask=m)
            plsc.store_compressed(rel_c.at[pl.ds(ptr, 16)], rel, mask=m)
            return ptr + jnp.sum(jnp.where(m, 1, 0).astype(jnp.int32))

        mseg = lax.fori_loop(0, ECH // 16, fbody, jnp.int32(0))

        ngrp = (mseg + (GK - 1)) // GK

        def pbody(j, _):
            off = mseg + j * 16
            src_c[pl.ds(off, 16)] = pad16
            rel_c[pl.ds(off, 16)] = trash16
            return 0

        lax.fori_loop(0, (ngrp * GK - mseg + 15) // 16, pbody, 0)

        @pl.when(ngrp > 0)
        def _():
            pltpu.async_copy(hs_hbm.at[src_c.at[pl.ds(0, GK)]],
                             gbuf.at[0], sem0)

        def gbody(k, _):
            p = lax.rem(k, 2)

            @pl.when((k + 1 < ngrp) & (p == 0))
            def _():
                pltpu.async_copy(hs_hbm.at[src_c.at[pl.ds((k + 1) * GK, GK)]],
                                 gbuf.at[1], sem1)

            @pl.when((k + 1 < ngrp) & (p == 1))
            def _():
                pltpu.async_copy(hs_hbm.at[src_c.at[pl.ds((k + 1) * GK, GK)]],
                                 gbuf.at[0], sem0)

            @pl.when(p == 0)
            def _():
                pltpu.make_async_copy(hs_hbm.at[src_c.at[pl.ds(0, GK)]],
                                      gbuf.at[0], sem0).wait()

            @pl.when(p == 1)
            def _():
                pltpu.make_async_copy(hs_hbm.at[src_c.at[pl.ds(0, GK)]],
                                      gbuf.at[1], sem1).wait()

            def group(g, _):
                relv = rel_c[pl.ds(k * GK + g * 16, 16)]
                for i in range(16):
                    rel = relv[i]
                    for j in range(D_OUT // 16):
                        plsc.addupdate(
                            acc.at[rel, pl.ds(j * 16, 16)],
                            gbuf[p, g * 16 + i, pl.ds(j * 16, 16)])
                return 0

            lax.fori_loop(0, GK // 16, group, 0)
            return 0

        lax.fori_loop(0, ngrp, gbody, 0)
        return 0

    lax.fori_loop(0, NSEG, segment, 0)

    @pl.when(s < 15)
    def _():
        pltpu.sync_copy(acc.at[pl.ds(0, _ROWS_A)],
                        tmp_hbm.at[pl.ds(base_row, _ROWS_A)])

    @pl.when(s == 15)
    def _():
        pltpu.sync_copy(acc.at[pl.ds(0, _ROWS_B)],
                        tmp_hbm.at[pl.ds(base_row, _ROWS_B)])


def _scatter_add(src_pad, dst_pad, hs):
    return pl.kernel(
        _scatter_body,
        out_type=jax.ShapeDtypeStruct((N, D_OUT), jnp.float32),
        mesh=_MESH,
        compiler_params=_SC_PARAMS,
        scratch_types=[
            pltpu.VMEM((ECH,), jnp.int32),
            pltpu.VMEM((ECH,), jnp.int32),
            pltpu.VMEM((SCAP,), jnp.int32),
            pltpu.VMEM((SCAP,), jnp.int32),
            pltpu.VMEM((2, GK, D_OUT), jnp.float32),
            pltpu.VMEM((ACC_ROWS, D_OUT), jnp.float32),
            pltpu.SemaphoreType.DMA,
            pltpu.SemaphoreType.DMA,
        ],
    )(src_pad, dst_pad, hs)


_BM = 2048


def _mm_body(x_ref, w_ref, dis_ref, o_ref):
    o_ref[...] = jnp.dot(x_ref[...], w_ref[...],
                         preferred_element_type=jnp.float32) * dis_ref[...]


def _matmul_scale(x, W, dis_col):
    M, K = x.shape
    _, Nc = W.shape
    return pl.pallas_call(
        _mm_body,
        grid=(pl.cdiv(M, _BM),),
        in_specs=[pl.BlockSpec((_BM, K), lambda i: (i, 0)),
                  pl.BlockSpec((K, Nc), lambda i: (0, 0)),
                  pl.BlockSpec((_BM, 1), lambda i: (i, 0))],
        out_specs=pl.BlockSpec((_BM, Nc), lambda i: (i, 0)),
        out_shape=jax.ShapeDtypeStruct((M, Nc), jnp.float32),
    )(x, W, dis_col)


def _elu_body(t_ref, dis_ref, b_ref, o_ref):
    v = t_ref[...] * dis_ref[...] + b_ref[...]
    o_ref[...] = jnp.where(v > 0, v, jnp.exp(v) - 1.0)


def _scale_bias_elu(tmp, dis_col, b2d):
    M, Nc = tmp.shape
    return pl.pallas_call(
        _elu_body,
        grid=(pl.cdiv(M, _BM),),
        in_specs=[pl.BlockSpec((_BM, Nc), lambda i: (i, 0)),
                  pl.BlockSpec((_BM, 1), lambda i: (i, 0)),
                  pl.BlockSpec((1, Nc), lambda i: (0, 0))],
        out_specs=pl.BlockSpec((_BM, Nc), lambda i: (i, 0)),
        out_shape=jax.ShapeDtypeStruct((M, Nc), jnp.float32),
    )(tmp, dis_col, b2d)


def kernel(x, edge_index, W, b):
    src = edge_index[0].astype(jnp.int32)
    dst = edge_index[1].astype(jnp.int32)
    pad_n = EPAD - E
    src_pad = jnp.concatenate([src, jnp.zeros((pad_n,), jnp.int32)])
    dst_pad = jnp.concatenate([dst, jnp.full((pad_n,), SENTINEL, jnp.int32)])

    parts = _deg_hist(dst_pad)
    dis = _dis_from_parts(parts)
    dis_col = dis[:N].reshape(N, 1)

    hs = _matmul_scale(x, W, dis_col)
    tmp = _scatter_add(src_pad, dst_pad, hs)
    return _scale_bias_elu(tmp, dis_col, b.reshape(1, D_OUT))
